# baseline (speedup 1.0000x reference)
import functools

import jax
import jax.numpy as jnp
from jax import lax
from jax.experimental import pallas as pl
from jax.experimental.pallas import tpu as pltpu
from jax.experimental.pallas import tpu_sc as plsc

D = 64
SCALE = 8.0
NC, NS = 2, 16
NW = NC * NS
CHUNK = 128
NBUF = 4
LANES = 16


def _emb_kernel(n_chunks, x_hbm, lut_hbm, out_hbm, idx_v, gbufs, sbufs,
                gsems, ssems):
    wid = lax.axis_index("s") * NC + lax.axis_index("c")
    per_w = n_chunks * CHUNK
    base = wid * per_w

    pltpu.sync_copy(x_hbm.at[wid], idx_v)

    def gather_start(j, b):
        pltpu.make_async_copy(lut_hbm.at[idx_v.at[j]], gbufs[b],
                              gsems[b]).start()

    def gather_wait(b):
        pltpu.make_async_copy(lut_hbm.at[idx_v.at[0]], gbufs[b],
                              gsems[b]).wait()

    def scatter_start(j, b):
        pltpu.make_async_copy(sbufs[b], out_hbm.at[pl.ds(base + j * CHUNK,
                                                         CHUNK)],
                              ssems[b]).start()

    def scatter_wait(b):
        pltpu.make_async_copy(sbufs[b], out_hbm.at[pl.ds(0, CHUNK)],
                              ssems[b]).wait()

    def scale_chunk(b):
        src, dst = gbufs[b], sbufs[b]

        def row(i, c):
            for k in range(D // LANES):
                sl = pl.ds(k * LANES, LANES)
                dst[i, sl] = src[i, sl] * SCALE
            return c

        lax.fori_loop(0, CHUNK, row, 0)

    n_groups = n_chunks // NBUF

    for b in range(NBUF):
        gather_start(b, b)

    for b in range(NBUF):
        gather_wait(b)
        scale_chunk(b)
        scatter_start(b, b)
        gather_start(NBUF + b, b)

    def group(g, c):
        for b in range(NBUF):
            j = g * NBUF + b
            gather_wait(b)
            scatter_wait(b)
            scale_chunk(b)
            scatter_start(j, b)
            gather_start(j + NBUF, b)
        return c

    lax.fori_loop(1, n_groups - 1, group, 0)

    for b in range(NBUF):
        j = (n_groups - 1) * NBUF + b
        gather_wait(b)
        scatter_wait(b)
        scale_chunk(b)
        scatter_start(j, b)

    for b in range(NBUF):
        scatter_wait(b)


def kernel(x, lut):
    b0, b1 = x.shape
    n = b0 * b1
    per_w = n // NW
    n_chunks = per_w // CHUNK
    x_r = x.reshape(NW, n_chunks, CHUNK).astype(jnp.int32)

    mesh = plsc.VectorSubcoreMesh(core_axis_name="c", subcore_axis_name="s",
                                  num_cores=NC, num_subcores=NS)
    run = functools.partial(
        pl.kernel,
        out_type=jax.ShapeDtypeStruct((n, D), jnp.float32),
        mesh=mesh,
        compiler_params=pltpu.CompilerParams(use_tc_tiling_on_sc=False),
        scratch_types=[
            pltpu.VMEM((n_chunks, CHUNK), jnp.int32),
            [pltpu.VMEM((CHUNK, D), jnp.float32) for _ in range(NBUF)],
            [pltpu.VMEM((CHUNK, D), jnp.float32) for _ in range(NBUF)],
            [pltpu.SemaphoreType.DMA for _ in range(NBUF)],
            [pltpu.SemaphoreType.DMA for _ in range(NBUF)],
        ],
    )(functools.partial(_emb_kernel, n_chunks))
    out = run(x_r, lut)
    return out.reshape(b0, b1, D)
